# baseline (speedup 1.0000x reference)
import jax
import jax.numpy as jnp
from jax.experimental import pallas as pl
from jax.experimental.pallas import tpu as pltpu

_WALL_COT = 0.5
_NSEG = 8


def _loss_body(out_ref, in_ref, m_ref, s_ref, e_ref, d_ref,
               loss_ref, cnt_acc, pos_acc, err_acc, recov_acc):
    b = pl.program_id(0)
    hc = pl.program_id(1)
    nb = pl.num_programs(0)
    nhc = pl.num_programs(1)

    @pl.when(jnp.logical_and(b == 0, hc == 0))
    def _init():
        cnt_acc[...] = jnp.zeros_like(cnt_acc)
        pos_acc[...] = jnp.zeros_like(pos_acc)
        err_acc[...] = jnp.zeros_like(err_acc)
        recov_acc[...] = jnp.zeros_like(recov_acc)

    m = m_ref[0, 0]
    o = out_ref[0]
    x = in_ref[0]
    t = jnp.where(m[None] >= _WALL_COT, 0.0, x)
    diff = o - t
    mse = jnp.sum(diff * diff, axis=0)
    mpos = m > 0.0
    recov_sum = jnp.sum(jnp.where(mpos, mse, 0.0), axis=0)
    recov_cnt = jnp.sum(mpos.astype(jnp.float32), axis=0)
    recov_acc[0:1, :] = recov_acc[0:1, :] + recov_sum[None]
    recov_acc[1:2, :] = recov_acc[1:2, :] + recov_cnt[None]

    e = e_ref[0]
    d = d_ref[0]
    ed = e - d
    re = jnp.sum(ed * ed, axis=0) / 96.0

    echunk, wechunk = e_ref.shape[2], e_ref.shape[3]
    hchunk, wchunk = m_ref.shape[2], m_ref.shape[3]
    he_i = jax.lax.broadcasted_iota(jnp.int32, (echunk, hchunk), 0)
    h_i = jax.lax.broadcasted_iota(jnp.int32, (echunk, hchunk), 1)
    p2 = (h_i == 4 * he_i).astype(jnp.float32)
    w_i = jax.lax.broadcasted_iota(jnp.int32, (wchunk, wechunk), 0)
    we_i = jax.lax.broadcasted_iota(jnp.int32, (wchunk, wechunk), 1)
    p1 = (w_i == 4 * we_i).astype(jnp.float32)

    seg = s_ref[0, 0]
    pm = jnp.logical_and(m < _WALL_COT, m > 0.0).astype(jnp.float32)
    seg_sub = jnp.dot(jnp.dot(p2, seg, preferred_element_type=jnp.float32),
                      p1, preferred_element_type=jnp.float32)
    pm_sub = jnp.dot(jnp.dot(p2, pm, preferred_element_type=jnp.float32),
                     p1, preferred_element_type=jnp.float32)

    cnt_rows = []
    pos_rows = []
    err_rows = []
    for s in range(_NSEG):
        ms = (seg_sub == float(s)).astype(jnp.float32)
        cnt_rows.append(jnp.sum(ms, axis=0)[None])
        pos_rows.append(jnp.sum(ms * pm_sub, axis=0)[None])
        err_rows.append(jnp.sum(ms * re, axis=0)[None])
    rows = pl.ds(b * _NSEG, _NSEG)
    cnt_acc[rows, :] = cnt_acc[rows, :] + jnp.concatenate(cnt_rows, axis=0)
    pos_acc[rows, :] = pos_acc[rows, :] + jnp.concatenate(pos_rows, axis=0)
    err_acc[rows, :] = err_acc[rows, :] + jnp.concatenate(err_rows, axis=0)

    @pl.when(jnp.logical_and(b == nb - 1, hc == nhc - 1))
    def _finish():
        cnt = jnp.sum(cnt_acc[...], axis=1, keepdims=True)
        pos = jnp.sum(pos_acc[...], axis=1, keepdims=True)
        err = jnp.sum(err_acc[...], axis=1, keepdims=True)
        valid = jnp.logical_not(cnt / 16384.0 < 0.01)
        mean_err = err / cnt
        flags = jnp.logical_and(valid, pos / cnt > 0.01)
        pos_sum = jnp.sum(jnp.where(flags, mean_err, 0.0))
        pos_cnt = jnp.sum(flags.astype(jnp.float32))
        rs = jnp.sum(recov_acc[0:1, :])
        rc = jnp.sum(recov_acc[1:2, :])
        loss = rs / rc + pos_sum / pos_cnt
        loss_ref[...] = jnp.broadcast_to(loss, loss_ref.shape)


def kernel(outputs, inputs, enc1, dec1, masks, segs, confidence,
           iteration, epoch):
    B, C, H, W = outputs.shape
    _, Ce, He, We = enc1.shape
    nhc = 8
    hchunk = H // nhc
    echunk = He // nhc

    grid = (B, nhc)
    loss_out = pl.pallas_call(
        _loss_body,
        grid=grid,
        in_specs=[
            pl.BlockSpec((1, C, hchunk, W), lambda b, h: (b, 0, h, 0)),
            pl.BlockSpec((1, C, hchunk, W), lambda b, h: (b, 0, h, 0)),
            pl.BlockSpec((1, 1, hchunk, W), lambda b, h: (b, 0, h, 0)),
            pl.BlockSpec((1, 1, hchunk, W), lambda b, h: (b, 0, h, 0)),
            pl.BlockSpec((1, Ce, echunk, We), lambda b, h: (b, 0, h, 0)),
            pl.BlockSpec((1, Ce, echunk, We), lambda b, h: (b, 0, h, 0)),
        ],
        out_specs=pl.BlockSpec((8, 128), lambda b, h: (0, 0)),
        out_shape=jax.ShapeDtypeStruct((8, 128), jnp.float32),
        scratch_shapes=[
            pltpu.VMEM((B * _NSEG, We), jnp.float32),
            pltpu.VMEM((B * _NSEG, We), jnp.float32),
            pltpu.VMEM((B * _NSEG, We), jnp.float32),
            pltpu.VMEM((8, W), jnp.float32),
        ],
        compiler_params=pltpu.CompilerParams(
            dimension_semantics=("arbitrary", "arbitrary")),
    )(outputs, inputs, masks, segs, enc1, dec1)
    return loss_out[0, 0]


# speedup vs baseline: 1.2617x; 1.2617x over previous
import jax
import jax.numpy as jnp
from jax.experimental import pallas as pl
from jax.experimental.pallas import tpu as pltpu

_WALL_COT = 0.5
_NSEG = 8


def _loss_body(out_ref, in_ref, m_ref, s_ref, e_ref, d_ref,
               loss_ref, cnt_acc, pos_acc, err_acc, recov_acc, re_acc):
    b = pl.program_id(0)
    c = pl.program_id(1)
    nb = pl.num_programs(0)
    nc = pl.num_programs(1)

    @pl.when(jnp.logical_and(b == 0, c == 0))
    def _init():
        cnt_acc[...] = jnp.zeros_like(cnt_acc)
        pos_acc[...] = jnp.zeros_like(pos_acc)
        err_acc[...] = jnp.zeros_like(err_acc)
        recov_acc[...] = jnp.zeros_like(recov_acc)

    @pl.when(c == 0)
    def _init_re():
        re_acc[...] = jnp.zeros_like(re_acc)

    m = m_ref[0]
    o = out_ref[0]
    x = in_ref[0]
    d = o - jnp.where(m >= _WALL_COT, 0.0, x)
    recov_sum = jnp.sum(jnp.where(m > 0.0, d * d, 0.0), axis=0)
    recov_acc[0:1, :] = recov_acc[0:1, :] + recov_sum[None]

    @pl.when(c == 0)
    def _cnt_pos():
        mcnt = jnp.sum((m > 0.0).astype(jnp.float32), axis=0)
        recov_acc[1:2, :] = recov_acc[1:2, :] + mcnt[None]

    e = e_ref[0]
    dd = d_ref[0]
    ed = e - dd
    sq = (ed * ed).reshape(-1, 128, 128)
    re_acc[...] = re_acc[...] + jnp.sum(sq, axis=0)

    @pl.when(c == nc - 1)
    def _seg_stats():
        he_i = jax.lax.broadcasted_iota(jnp.int32, (128, 512), 0)
        h_i = jax.lax.broadcasted_iota(jnp.int32, (128, 512), 1)
        p2 = (h_i == 4 * he_i).astype(jnp.float32)
        w_i = jax.lax.broadcasted_iota(jnp.int32, (512, 128), 0)
        we_i = jax.lax.broadcasted_iota(jnp.int32, (512, 128), 1)
        p1 = (w_i == 4 * we_i).astype(jnp.float32)

        seg = s_ref[0]
        pm = jnp.logical_and(m < _WALL_COT, m > 0.0).astype(jnp.float32)
        seg_sub = jnp.dot(
            jnp.dot(p2, seg, preferred_element_type=jnp.float32),
            p1, preferred_element_type=jnp.float32)
        pm_sub = jnp.dot(
            jnp.dot(p2, pm, preferred_element_type=jnp.float32),
            p1, preferred_element_type=jnp.float32)
        re = re_acc[...] / 96.0

        cnt_rows = []
        pos_rows = []
        err_rows = []
        for s in range(_NSEG):
            ms = (seg_sub == float(s)).astype(jnp.float32)
            cnt_rows.append(jnp.sum(ms, axis=0)[None])
            pos_rows.append(jnp.sum(ms * pm_sub, axis=0)[None])
            err_rows.append(jnp.sum(ms * re, axis=0)[None])
        rows = pl.ds(b * _NSEG, _NSEG)
        cnt_acc[rows, :] = cnt_acc[rows, :] + jnp.concatenate(cnt_rows, 0)
        pos_acc[rows, :] = pos_acc[rows, :] + jnp.concatenate(pos_rows, 0)
        err_acc[rows, :] = err_acc[rows, :] + jnp.concatenate(err_rows, 0)

    @pl.when(jnp.logical_and(b == nb - 1, c == nc - 1))
    def _finish():
        cnt = jnp.sum(cnt_acc[...], axis=1, keepdims=True)
        pos = jnp.sum(pos_acc[...], axis=1, keepdims=True)
        err = jnp.sum(err_acc[...], axis=1, keepdims=True)
        valid = jnp.logical_not(cnt / 16384.0 < 0.01)
        mean_err = err / cnt
        flags = jnp.logical_and(valid, pos / cnt > 0.01)
        pos_sum = jnp.sum(jnp.where(flags, mean_err, 0.0))
        pos_cnt = jnp.sum(flags.astype(jnp.float32))
        rs = jnp.sum(recov_acc[0:1, :])
        rc = jnp.sum(recov_acc[1:2, :])
        loss = rs / rc + pos_sum / pos_cnt
        loss_ref[...] = jnp.broadcast_to(loss, loss_ref.shape)


def kernel(outputs, inputs, enc1, dec1, masks, segs, confidence,
           iteration, epoch):
    B, C, H, W = outputs.shape
    _, Ce, He, We = enc1.shape
    outputs3 = outputs.reshape(B, C * H, W)
    inputs3 = inputs.reshape(B, C * H, W)
    masks3 = masks.reshape(B, H, W)
    segs3 = segs.reshape(B, H, W)
    enc3 = enc1.reshape(B, Ce * He, We)
    dec3 = dec1.reshape(B, Ce * He, We)
    eslab = Ce * He // C

    grid = (B, C)
    loss_out = pl.pallas_call(
        _loss_body,
        grid=grid,
        in_specs=[
            pl.BlockSpec((1, H, W), lambda b, c: (b, c, 0)),
            pl.BlockSpec((1, H, W), lambda b, c: (b, c, 0)),
            pl.BlockSpec((1, H, W), lambda b, c: (b, 0, 0)),
            pl.BlockSpec((1, H, W), lambda b, c: (b, 0, 0)),
            pl.BlockSpec((1, eslab, We), lambda b, c: (b, c, 0)),
            pl.BlockSpec((1, eslab, We), lambda b, c: (b, c, 0)),
        ],
        out_specs=pl.BlockSpec((8, 128), lambda b, c: (0, 0)),
        out_shape=jax.ShapeDtypeStruct((8, 128), jnp.float32),
        scratch_shapes=[
            pltpu.VMEM((B * _NSEG, We), jnp.float32),
            pltpu.VMEM((B * _NSEG, We), jnp.float32),
            pltpu.VMEM((B * _NSEG, We), jnp.float32),
            pltpu.VMEM((8, W), jnp.float32),
            pltpu.VMEM((He, We), jnp.float32),
        ],
        compiler_params=pltpu.CompilerParams(
            dimension_semantics=("arbitrary", "arbitrary")),
    )(outputs3, inputs3, masks3, segs3, enc3, dec3)
    return loss_out[0, 0]


# speedup vs baseline: 1.3284x; 1.0529x over previous
import jax
import jax.numpy as jnp
from jax.experimental import pallas as pl
from jax.experimental.pallas import tpu as pltpu

_WALL_COT = 0.5
_NSEG = 8


def _loss_body(out_ref, in_ref, m_ref, s_ref, e_ref, d_ref,
               loss_ref, cnt_acc, pos_acc, err_acc, recov_acc):
    b = pl.program_id(0)
    hc = pl.program_id(1)
    nb = pl.num_programs(0)
    nhc = pl.num_programs(1)

    @pl.when(jnp.logical_and(b == 0, hc == 0))
    def _init():
        cnt_acc[...] = jnp.zeros_like(cnt_acc)
        pos_acc[...] = jnp.zeros_like(pos_acc)
        err_acc[...] = jnp.zeros_like(err_acc)
        recov_acc[...] = jnp.zeros_like(recov_acc)

    m = m_ref[0, 0]
    o = out_ref[0]
    x = in_ref[0]
    t = jnp.where(m[None] >= _WALL_COT, 0.0, x)
    diff = o - t
    mse = jnp.sum(diff * diff, axis=0)
    mpos = m > 0.0
    recov_sum = jnp.sum(jnp.where(mpos, mse, 0.0), axis=0)
    recov_cnt = jnp.sum(mpos.astype(jnp.float32), axis=0)
    recov_acc[0:1, :] = recov_acc[0:1, :] + recov_sum[None]
    recov_acc[1:2, :] = recov_acc[1:2, :] + recov_cnt[None]

    e = e_ref[0]
    d = d_ref[0]
    ed = e - d
    re = jnp.sum(ed * ed, axis=0) / 96.0

    echunk, wechunk = e_ref.shape[2], e_ref.shape[3]
    hchunk, wchunk = m_ref.shape[2], m_ref.shape[3]
    he_i = jax.lax.broadcasted_iota(jnp.int32, (echunk, hchunk), 0)
    h_i = jax.lax.broadcasted_iota(jnp.int32, (echunk, hchunk), 1)
    p2 = (h_i == 4 * he_i).astype(jnp.float32)
    w_i = jax.lax.broadcasted_iota(jnp.int32, (wchunk, wechunk), 0)
    we_i = jax.lax.broadcasted_iota(jnp.int32, (wchunk, wechunk), 1)
    p1 = (w_i == 4 * we_i).astype(jnp.float32)

    seg = s_ref[0, 0]
    pm = jnp.logical_and(m < _WALL_COT, m > 0.0).astype(jnp.float32)
    seg_sub = jnp.dot(jnp.dot(p2, seg, preferred_element_type=jnp.float32),
                      p1, preferred_element_type=jnp.float32)
    pm_sub = jnp.dot(jnp.dot(p2, pm, preferred_element_type=jnp.float32),
                     p1, preferred_element_type=jnp.float32)

    cnt_rows = []
    pos_rows = []
    err_rows = []
    for s in range(_NSEG):
        ms = (seg_sub == float(s)).astype(jnp.float32)
        cnt_rows.append(jnp.sum(ms, axis=0)[None])
        pos_rows.append(jnp.sum(ms * pm_sub, axis=0)[None])
        err_rows.append(jnp.sum(ms * re, axis=0)[None])
    rows = pl.ds(b * _NSEG, _NSEG)
    cnt_acc[rows, :] = cnt_acc[rows, :] + jnp.concatenate(cnt_rows, axis=0)
    pos_acc[rows, :] = pos_acc[rows, :] + jnp.concatenate(pos_rows, axis=0)
    err_acc[rows, :] = err_acc[rows, :] + jnp.concatenate(err_rows, axis=0)

    @pl.when(jnp.logical_and(b == nb - 1, hc == nhc - 1))
    def _finish():
        cnt = jnp.sum(cnt_acc[...], axis=1, keepdims=True)
        pos = jnp.sum(pos_acc[...], axis=1, keepdims=True)
        err = jnp.sum(err_acc[...], axis=1, keepdims=True)
        valid = jnp.logical_not(cnt / 16384.0 < 0.01)
        mean_err = err / cnt
        flags = jnp.logical_and(valid, pos / cnt > 0.01)
        pos_sum = jnp.sum(jnp.where(flags, mean_err, 0.0))
        pos_cnt = jnp.sum(flags.astype(jnp.float32))
        rs = jnp.sum(recov_acc[0:1, :])
        rc = jnp.sum(recov_acc[1:2, :])
        loss = rs / rc + pos_sum / pos_cnt
        loss_ref[...] = jnp.broadcast_to(loss, loss_ref.shape)


def kernel(outputs, inputs, enc1, dec1, masks, segs, confidence,
           iteration, epoch):
    B, C, H, W = outputs.shape
    _, Ce, He, We = enc1.shape
    nhc = 4
    hchunk = H // nhc
    echunk = He // nhc

    grid = (B, nhc)
    loss_out = pl.pallas_call(
        _loss_body,
        grid=grid,
        in_specs=[
            pl.BlockSpec((1, C, hchunk, W), lambda b, h: (b, 0, h, 0)),
            pl.BlockSpec((1, C, hchunk, W), lambda b, h: (b, 0, h, 0)),
            pl.BlockSpec((1, 1, hchunk, W), lambda b, h: (b, 0, h, 0)),
            pl.BlockSpec((1, 1, hchunk, W), lambda b, h: (b, 0, h, 0)),
            pl.BlockSpec((1, Ce, echunk, We), lambda b, h: (b, 0, h, 0)),
            pl.BlockSpec((1, Ce, echunk, We), lambda b, h: (b, 0, h, 0)),
        ],
        out_specs=pl.BlockSpec((8, 128), lambda b, h: (0, 0)),
        out_shape=jax.ShapeDtypeStruct((8, 128), jnp.float32),
        scratch_shapes=[
            pltpu.VMEM((B * _NSEG, We), jnp.float32),
            pltpu.VMEM((B * _NSEG, We), jnp.float32),
            pltpu.VMEM((B * _NSEG, We), jnp.float32),
            pltpu.VMEM((8, W), jnp.float32),
        ],
        compiler_params=pltpu.CompilerParams(
            dimension_semantics=("arbitrary", "arbitrary")),
    )(outputs, inputs, masks, segs, enc1, dec1)
    return loss_out[0, 0]
